# baseline (device time: 10255 ns/iter reference)
import jax
import jax.numpy as jnp
from jax import lax
from jax.experimental import pallas as pl
from jax.experimental.pallas import tpu as pltpu

N_DEV = 8
N_CHUNKS = 4


def kernel(x):
    m_per, n = x.shape
    chunk_m = m_per // N_CHUNKS
    assert chunk_m * N_CHUNKS == m_per

    def body(x_ref, out_ref, chunk_ref, local_ref, acc_ref,
             load_sems, send_sems, recv_sems):
        my = lax.axis_index("i")

        barrier_sem = pltpu.get_barrier_semaphore()
        for d in range(1, N_DEV):
            tgt = lax.rem(my + d, N_DEV)
            pl.semaphore_signal(
                barrier_sem, inc=1,
                device_id=(tgt,), device_id_type=pl.DeviceIdType.MESH,
            )

        copies = []
        for c in range(N_CHUNKS):
            cp = pltpu.make_async_copy(
                x_ref.at[pl.ds(c * chunk_m, chunk_m), :],
                chunk_ref.at[c % 3],
                load_sems.at[c % 3],
            )
            copies.append(cp)
        copies[0].start()
        copies[1].start()
        acc = None
        for c in range(N_CHUNKS):
            copies[c].wait()
            if c + 2 < N_CHUNKS:
                copies[c + 2].start()
            part = jnp.max(chunk_ref[c % 3], axis=0, keepdims=True)
            acc = part if acc is None else jnp.maximum(acc, part)
        local_ref[:, :] = acc
        acc_ref[pl.ds(my, 1), :] = acc

        pl.semaphore_wait(barrier_sem, N_DEV - 1)

        rdmas = []
        for d in range(1, N_DEV):
            tgt = lax.rem(my + d, N_DEV)
            rdma = pltpu.make_async_remote_copy(
                src_ref=local_ref,
                dst_ref=acc_ref.at[pl.ds(my, 1), :],
                send_sem=send_sems.at[d],
                recv_sem=recv_sems.at[d],
                device_id=(tgt,),
                device_id_type=pl.DeviceIdType.MESH,
            )
            rdma.start()
            rdmas.append(rdma)
        for rdma in rdmas:
            rdma.wait()

        out_ref[:, :] = jnp.max(acc_ref[:, :], axis=0, keepdims=True)

    return pl.pallas_call(
        body,
        out_shape=jax.ShapeDtypeStruct((1, n), x.dtype),
        in_specs=[pl.BlockSpec(memory_space=pltpu.MemorySpace.HBM)],
        out_specs=pl.BlockSpec(memory_space=pltpu.VMEM),
        scratch_shapes=[
            pltpu.VMEM((3, chunk_m, n), x.dtype),
            pltpu.VMEM((1, n), x.dtype),
            pltpu.VMEM((N_DEV, n), x.dtype),
            pltpu.SemaphoreType.DMA((3,)),
            pltpu.SemaphoreType.DMA((N_DEV,)),
            pltpu.SemaphoreType.DMA((N_DEV,)),
        ],
        compiler_params=pltpu.CompilerParams(collective_id=0),
    )(x)


# device time: 9120 ns/iter; 1.1245x vs baseline; 1.1245x over previous
import jax
import jax.numpy as jnp
from jax import lax
from jax.experimental import pallas as pl
from jax.experimental.pallas import tpu as pltpu

N_DEV = 8
N_CHUNKS = 8


def kernel(x):
    m_per, n = x.shape
    chunk_m = m_per // N_CHUNKS
    assert chunk_m * N_CHUNKS == m_per

    def body(x_ref, out_ref, chunks_ref, local_ref, acc_ref,
             load_sems, send_sems, recv_sems):
        my = lax.axis_index("i")

        barrier_sem = pltpu.get_barrier_semaphore()
        for d in range(1, N_DEV):
            tgt = lax.rem(my + d, N_DEV)
            pl.semaphore_signal(
                barrier_sem, inc=1,
                device_id=(tgt,), device_id_type=pl.DeviceIdType.MESH,
            )

        copies = []
        for c in range(N_CHUNKS):
            cp = pltpu.make_async_copy(
                x_ref.at[pl.ds(c * chunk_m, chunk_m), :],
                chunks_ref.at[c],
                load_sems.at[c],
            )
            cp.start()
            copies.append(cp)

        acc = None
        for c in range(N_CHUNKS):
            copies[c].wait()
            part = jnp.max(chunks_ref[c], axis=0, keepdims=True)
            acc = part if acc is None else jnp.maximum(acc, part)
        local_ref[:, :] = acc

        pl.semaphore_wait(barrier_sem, N_DEV - 1)

        rdmas = []
        for d in range(1, N_DEV):
            tgt = lax.rem(my + d, N_DEV)
            rdma = pltpu.make_async_remote_copy(
                src_ref=local_ref,
                dst_ref=acc_ref.at[pl.ds(my, 1), :],
                send_sem=send_sems.at[d],
                recv_sem=recv_sems.at[d],
                device_id=(tgt,),
                device_id_type=pl.DeviceIdType.MESH,
            )
            rdma.start()
            rdmas.append(rdma)
        acc_ref[pl.ds(my, 1), :] = local_ref[:, :]
        for rdma in rdmas:
            rdma.wait()

        local_ref[:, :] = jnp.max(acc_ref[:, :], axis=0, keepdims=True)
        out_cp = pltpu.make_async_copy(local_ref, out_ref, load_sems.at[0])
        out_cp.start()
        out_cp.wait()

    x = pltpu.with_memory_space_constraint(x, pltpu.MemorySpace.HBM)
    return pl.pallas_call(
        body,
        out_shape=jax.ShapeDtypeStruct((1, n), x.dtype),
        in_specs=[pl.BlockSpec(memory_space=pl.ANY)],
        out_specs=pl.BlockSpec(memory_space=pltpu.MemorySpace.HBM),
        scratch_shapes=[
            pltpu.VMEM((N_CHUNKS, chunk_m, n), x.dtype),
            pltpu.VMEM((1, n), x.dtype),
            pltpu.VMEM((N_DEV, n), x.dtype),
            pltpu.SemaphoreType.DMA((N_CHUNKS,)),
            pltpu.SemaphoreType.DMA((N_DEV,)),
            pltpu.SemaphoreType.DMA((N_DEV,)),
        ],
        compiler_params=pltpu.CompilerParams(collective_id=0),
    )(x)
